# Initial kernel scaffold; baseline (speedup 1.0000x reference)
import jax
import jax.numpy as jnp
from jax.experimental import pallas as pl


def kernel(feat, edge_index, logits, tau_1, tau_2, Wy, by):
    raise NotImplementedError("write your pallas kernel here")


# speedup vs baseline: 11.3873x; 11.3873x over previous
import functools

import jax
import jax.numpy as jnp
from jax import lax
from jax.experimental import pallas as pl
from jax.experimental.pallas import tpu as pltpu
from jax.experimental.pallas import tpu_sc as plsc

N = 10000
E = 320000
D = 128
C = 40
K = 3

NC = 2
NS = 16
NW = NC * NS
EW = E // NW
CH = 80
NCHUNK = EW // CH

N2 = 10240
NSL = N2 // NS
NCF = N * C
CSL = NCF // NS


def _fill(ref, n16, value):
    def body(i, _):
        ref[pl.ds(i * 16, 16)] = jnp.full((16,), value, ref.dtype)
        return 0
    lax.fori_loop(0, n16, body, 0)


def _fill2d(ref, value):
    r = ref.shape[0]
    def body(i, _):
        ref[i // 8, pl.ds((i % 8) * 16, 16)] = jnp.full((16,), value, ref.dtype)
        return 0
    lax.fori_loop(0, r * 8, body, 0)


@functools.cache
def _sc_kernels():
    mesh = plsc.VectorSubcoreMesh(core_axis_name="c", subcore_axis_name="s",
                                  num_cores=NC, num_subcores=NS)

    sc_degree = functools.partial(
        pl.kernel,
        out_type=jax.ShapeDtypeStruct((NC * N2,), jnp.float32),
        mesh=mesh,
        scratch_types=[
            pltpu.VMEM_SHARED((N2,), jnp.float32),
            pltpu.VMEM((CH,), jnp.int32),
            pltpu.VMEM((CH,), jnp.float32),
            pltpu.VMEM((N2 // NS,), jnp.float32),
        ],
    )(_sc_degree_body)

    sc_edges = functools.partial(
        pl.kernel,
        out_type=(jax.ShapeDtypeStruct((NC * N2, D), jnp.float32),
                  jax.ShapeDtypeStruct((NC * NCF,), jnp.float32)),
        mesh=mesh,
        scratch_types=[
            pltpu.VMEM_SHARED((N2, D), jnp.float32),
            pltpu.VMEM_SHARED((NCF,), jnp.float32),
            pltpu.VMEM((CH,), jnp.int32),
            pltpu.VMEM((CH,), jnp.int32),
            pltpu.VMEM((CH,), jnp.int32),
            pltpu.VMEM((CH,), jnp.int32),
            pltpu.VMEM((CH,), jnp.float32),
            pltpu.VMEM((CH, D), jnp.float32),
            pltpu.VMEM((64, D), jnp.float32),
            pltpu.VMEM((1008,), jnp.float32),
            pltpu.SemaphoreType.DMA,
            pltpu.SemaphoreType.DMA,
        ],
    )(_sc_edges_body)

    return sc_degree, sc_edges


def _sc_degree_body(dst_hbm, out_hbm, deg_sh, dst_v, ones_v, z_v):
    c = lax.axis_index("c")
    s = lax.axis_index("s")
    wid = c * NS + s
    _fill(ones_v, CH // 16, 1.0)
    _fill(z_v, (N2 // NS) // 16, 0.0)
    pltpu.sync_copy(z_v, deg_sh.at[pl.ds(s * (N2 // NS), N2 // NS)])
    plsc.subcore_barrier()
    wbase = wid * EW

    def body(i, _):
        base = pl.multiple_of(wbase + i * CH, 8)
        pltpu.sync_copy(dst_hbm.at[pl.ds(base, CH)], dst_v)
        pltpu.sync_copy(ones_v, deg_sh.at[dst_v], add=True)
        return 0

    lax.fori_loop(0, NCHUNK, body, 0)
    plsc.subcore_barrier()
    obase = pl.multiple_of(c * N2 + s * (N2 // NS), 8)
    pltpu.sync_copy(deg_sh.at[pl.ds(s * (N2 // NS), N2 // NS)], z_v)
    pltpu.sync_copy(z_v, out_hbm.at[pl.ds(obase, N2 // NS)])


def _sc_edges_body(feat_hbm, cp_hbm, src_hbm, dst_hbm, agg_out, cnt_out,
                   agg_sh, cnt_sh, src_v, dst_v, cps_v, fidx_v, ones_v,
                   rows_v, zrows_v, zcnt_v, sem1, sem2):
    c = lax.axis_index("c")
    s = lax.axis_index("s")
    wid = c * NS + s
    _fill(ones_v, CH // 16, 1.0)
    _fill2d(zrows_v, 0.0)
    _fill(zcnt_v, 63, 0.0)
    for j in range(10):
        pltpu.sync_copy(zrows_v, agg_sh.at[pl.ds(s * NSL + j * 64, 64), :])
    for j in range(25):
        pltpu.sync_copy(zcnt_v.at[pl.ds(0, 1000)],
                        cnt_sh.at[pl.ds(s * CSL + j * 1000, 1000)])
    plsc.subcore_barrier()
    wbase = wid * EW

    def body(i, _):
        base = pl.multiple_of(wbase + i * CH, 8)
        pltpu.sync_copy(src_hbm.at[pl.ds(base, CH)], src_v)
        pltpu.sync_copy(dst_hbm.at[pl.ds(base, CH)], dst_v)
        cp_dma = pltpu.async_copy(cp_hbm.at[src_v], cps_v, sem1)
        row_dma = pltpu.async_copy(feat_hbm.at[src_v], rows_v, sem2)
        cp_dma.wait()
        for j in range(CH // 16):
            d16 = dst_v[pl.ds(j * 16, 16)]
            c16 = cps_v[pl.ds(j * 16, 16)]
            fidx_v[pl.ds(j * 16, 16)] = d16 * C + c16
        pltpu.sync_copy(ones_v, cnt_sh.at[fidx_v], add=True)
        row_dma.wait()
        pltpu.sync_copy(rows_v, agg_sh.at[dst_v], add=True)
        return 0

    lax.fori_loop(0, NCHUNK, body, 0)
    plsc.subcore_barrier()
    for j in range(10):
        pltpu.sync_copy(agg_sh.at[pl.ds(s * NSL + j * 64, 64), :], zrows_v)
        pltpu.sync_copy(zrows_v,
                        agg_out.at[pl.ds(c * N2 + s * NSL + j * 64, 64), :])
    for j in range(25):
        pltpu.sync_copy(cnt_sh.at[pl.ds(s * CSL + j * 1000, 1000)],
                        zcnt_v.at[pl.ds(0, 1000)])
        obase = pl.multiple_of(c * NCF + s * CSL + j * 1000, 8)
        pltpu.sync_copy(zcnt_v.at[pl.ds(0, 1000)],
                        cnt_out.at[pl.ds(obase, 1000)])


def _argmax_rows(x):
    m = jnp.max(x, axis=1, keepdims=True)
    ci = lax.broadcasted_iota(jnp.int32, x.shape, 1)
    return jnp.min(jnp.where(x == m, ci, x.shape[1]), axis=1, keepdims=True)


def _tc_pre_body(degp_ref, feat_ref, logits_ref, fs_ref, cp_ref,
                 norm_ref, deg_ref):
    indeg = degp_ref[0, 0] + degp_ref[1, 0]
    deg = jnp.maximum(indeg, 1.0)
    norm = 1.0 / jnp.sqrt(deg)
    deg_ref[0] = deg
    norm_ref[0] = norm
    fs_ref[...] = feat_ref[...] * norm
    cp_ref[0] = _argmax_rows(logits_ref[...])


def _tc_gate_body(cntp_ref, cp_ref, deg_ref, t1_ref, t2_ref, z_ref):
    counts = cntp_ref[0] + cntp_ref[1]
    deg = deg_ref[...]
    cp = cp_ref[...]
    ci = lax.broadcasted_iota(jnp.int32, counts.shape, 1)
    f1 = jnp.sum(jnp.where(ci == cp, counts, 0.0), axis=1,
                 keepdims=True) / deg
    present = jnp.sum(counts, axis=0, keepdims=True) > 0
    p = jnp.maximum(counts / deg, 1e-5)
    f2 = -jnp.sum(jnp.where(present, p * jnp.log(p), 0.0), axis=1,
                  keepdims=True)
    m1 = jnp.mean(f1)
    v1 = jnp.mean((f1 - m1) ** 2)
    nf1 = (f1 - m1) / jnp.sqrt(v1 + 1e-5)
    m2 = jnp.mean(f2)
    v2 = jnp.mean((f2 - m2) ** 2)
    nf2 = (f2 - m2) / jnp.sqrt(v2 + 1e-5)
    a = -(nf1 - t1_ref[0, 0])
    b = -(nf2 - t2_ref[0, 0])
    z_ref[...] = (1.0 / (1.0 + jnp.exp(-a))) * (1.0 / (1.0 + jnp.exp(-b)))


def _tc_update_body(z_ref, norm_ref, aggp_ref, fs_ref, wyt_ref, by_ref,
                    fnew_ref, fsn_ref, cpn_ref):
    agg = aggp_ref[0] + aggp_ref[1]
    z = z_ref[0]
    norm = norm_ref[0]
    fnew = z * (agg * norm) + fs_ref[...]
    fnew_ref[...] = fnew
    fsn = fnew * norm
    fsn_ref[...] = fsn
    logits = jnp.dot(fsn, wyt_ref[...],
                     preferred_element_type=jnp.float32) + by_ref[...]
    cpn_ref[0] = _argmax_rows(logits)


_GRID = 10
_BN = N // _GRID


def _wide_spec(width):
    return pl.BlockSpec((_BN, width), lambda i: (i, 0))


def _skinny_spec():
    return pl.BlockSpec((1, _BN, 1), lambda i: (i, 0, 0))


def _skinny_shape(dtype):
    return jax.ShapeDtypeStruct((_GRID, _BN, 1), dtype)


def _tc_pre(degp, feat, logits):
    return pl.pallas_call(
        _tc_pre_body,
        grid=(_GRID,),
        in_specs=[
            pl.BlockSpec((2, 1, _BN, 1), lambda i: (0, i, 0, 0)),
            _wide_spec(D),
            _wide_spec(C),
        ],
        out_specs=[_wide_spec(D), _skinny_spec(), _skinny_spec(),
                   _skinny_spec()],
        out_shape=[
            jax.ShapeDtypeStruct((N, D), jnp.float32),
            _skinny_shape(jnp.int32),
            _skinny_shape(jnp.float32),
            _skinny_shape(jnp.float32),
        ],
    )(degp, feat, logits)


def _tc_gate(cntp, cp, deg, t1, t2):
    return pl.pallas_call(
        _tc_gate_body,
        out_shape=jax.ShapeDtypeStruct((N, 1), jnp.float32),
    )(cntp, cp, deg, t1, t2)


def _tc_update(z, norm, aggp, fs, wyt, by2):
    return pl.pallas_call(
        _tc_update_body,
        grid=(_GRID,),
        in_specs=[
            _skinny_spec(),
            _skinny_spec(),
            pl.BlockSpec((2, _BN, D), lambda i: (0, i, 0)),
            _wide_spec(D),
            pl.BlockSpec((D, C), lambda i: (0, 0)),
            pl.BlockSpec((1, C), lambda i: (0, 0)),
        ],
        out_specs=[_wide_spec(D), _wide_spec(D), _skinny_spec()],
        out_shape=[
            jax.ShapeDtypeStruct((N, D), jnp.float32),
            jax.ShapeDtypeStruct((N, D), jnp.float32),
            _skinny_shape(jnp.int32),
        ],
    )(z, norm, aggp, fs, wyt, by2)


def kernel(feat, edge_index, logits, tau_1, tau_2, Wy, by):
    src = edge_index[0]
    dst = edge_index[1]
    wyt = Wy.T
    by2 = by.reshape(1, C)
    t1 = tau_1.reshape(1, 1)
    t2 = tau_2.reshape(1, 1)

    sc_degree, sc_edges = _sc_kernels()
    degp = sc_degree(dst)
    degp = degp.reshape(NC, N2)[:, :N].reshape(NC, _GRID, _BN, 1)
    fs, cp_b, norm_b, deg_b = _tc_pre(degp, feat, logits)
    deg = deg_b.reshape(N, 1)

    fnew = feat
    for _ in range(K):
        agg2, cnt2 = sc_edges(fs, cp_b.reshape(N), src, dst)
        cntp = cnt2.reshape(NC, N, C)
        aggp = agg2.reshape(NC, N2, D)[:, :N, :]
        z = _tc_gate(cntp, cp_b.reshape(N, 1), deg, t1, t2)
        fnew, fs, cp_b = _tc_update(z.reshape(_GRID, _BN, 1), norm_b,
                                    aggp, fs, wyt, by2)
    return fnew


# speedup vs baseline: 15.7532x; 1.3834x over previous
import functools

import jax
import jax.numpy as jnp
from jax import lax
from jax.experimental import pallas as pl
from jax.experimental.pallas import tpu as pltpu
from jax.experimental.pallas import tpu_sc as plsc

N = 10000
E = 320000
D = 128
C = 40
K = 3

NC = 2
NS = 16
NW = NC * NS
EW = E // NW
CH = 80
NCHUNK = EW // CH

N2 = 10240
NSL = N2 // NS
NCF = N * C
CSL = NCF // NS


def _fill(ref, n16, value):
    def body(i, _):
        ref[pl.ds(i * 16, 16)] = jnp.full((16,), value, ref.dtype)
        return 0
    lax.fori_loop(0, n16, body, 0)


def _fill2d(ref, value):
    r = ref.shape[0]
    def body(i, _):
        ref[i // 8, pl.ds((i % 8) * 16, 16)] = jnp.full((16,), value, ref.dtype)
        return 0
    lax.fori_loop(0, r * 8, body, 0)


@functools.cache
def _sc_kernels():
    mesh = plsc.VectorSubcoreMesh(core_axis_name="c", subcore_axis_name="s",
                                  num_cores=NC, num_subcores=NS)

    sc_degree = functools.partial(
        pl.kernel,
        out_type=jax.ShapeDtypeStruct((NC * N2,), jnp.float32),
        mesh=mesh,
        scratch_types=[
            pltpu.VMEM_SHARED((N2,), jnp.float32),
            pltpu.VMEM((CH,), jnp.int32),
            pltpu.VMEM((CH,), jnp.float32),
            pltpu.VMEM((N2 // NS,), jnp.float32),
        ],
    )(_sc_degree_body)

    sc_edges = functools.partial(
        pl.kernel,
        out_type=(jax.ShapeDtypeStruct((NC * N2, D), jnp.float32),
                  jax.ShapeDtypeStruct((NC * NCF,), jnp.float32)),
        mesh=mesh,
        scratch_types=[
            pltpu.VMEM_SHARED((N2, D), jnp.float32),
            pltpu.VMEM_SHARED((NCF,), jnp.float32),
            pltpu.VMEM((2, CH), jnp.int32),
            pltpu.VMEM((2, CH), jnp.int32),
            pltpu.VMEM((2, CH), jnp.int32),
            pltpu.VMEM((CH,), jnp.int32),
            pltpu.VMEM((CH,), jnp.float32),
            pltpu.VMEM((2, CH, D), jnp.float32),
            pltpu.VMEM((1008,), jnp.float32),
            pltpu.SemaphoreType.DMA,
            pltpu.SemaphoreType.DMA,
            pltpu.SemaphoreType.DMA,
            pltpu.SemaphoreType.DMA,
        ],
    )(_sc_edges_body)

    return sc_degree, sc_edges


def _sc_degree_body(dst_hbm, out_hbm, deg_sh, dst_v, ones_v, z_v):
    c = lax.axis_index("c")
    s = lax.axis_index("s")
    wid = c * NS + s
    _fill(ones_v, CH // 16, 1.0)
    _fill(z_v, (N2 // NS) // 16, 0.0)
    pltpu.sync_copy(z_v, deg_sh.at[pl.ds(s * (N2 // NS), N2 // NS)])
    plsc.subcore_barrier()
    wbase = wid * EW

    def body(i, _):
        base = pl.multiple_of(wbase + i * CH, 8)
        pltpu.sync_copy(dst_hbm.at[pl.ds(base, CH)], dst_v)
        pltpu.sync_copy(ones_v, deg_sh.at[dst_v], add=True)
        return 0

    lax.fori_loop(0, NCHUNK, body, 0)
    plsc.subcore_barrier()
    obase = pl.multiple_of(c * N2 + s * (N2 // NS), 8)
    pltpu.sync_copy(deg_sh.at[pl.ds(s * (N2 // NS), N2 // NS)], z_v)
    pltpu.sync_copy(z_v, out_hbm.at[pl.ds(obase, N2 // NS)])


def _sc_edges_body(feat_hbm, cp_hbm, src_hbm, dst_hbm, agg_out, cnt_out,
                   agg_sh, cnt_sh, src_v, dst_v, cps_v, fidx_v, ones_v,
                   rows_v, zcnt_v, csA, rsA, csB, rsB):
    c = lax.axis_index("c")
    s = lax.axis_index("s")
    wid = c * NS + s
    _fill(ones_v, CH // 16, 1.0)
    _fill2d(rows_v.at[0], 0.0)
    _fill(zcnt_v, 63, 0.0)
    for j in range(8):
        pltpu.sync_copy(rows_v.at[0], agg_sh.at[pl.ds(s * NSL + j * CH, CH), :])
    for j in range(25):
        pltpu.sync_copy(zcnt_v.at[pl.ds(0, 1000)],
                        cnt_sh.at[pl.ds(s * CSL + j * 1000, 1000)])
    plsc.subcore_barrier()
    wbase = wid * EW
    sems = ((csA, rsA), (csB, rsB))

    def fire(chunk, b):
        base = pl.multiple_of(wbase + chunk * CH, 8)
        pltpu.sync_copy(src_hbm.at[pl.ds(base, CH)], src_v.at[b])
        pltpu.sync_copy(dst_hbm.at[pl.ds(base, CH)], dst_v.at[b])
        pltpu.async_copy(cp_hbm.at[src_v.at[b]], cps_v.at[b], sems[b][0])
        pltpu.async_copy(feat_hbm.at[src_v.at[b]], rows_v.at[b], sems[b][1])

    def drain(b):
        pltpu.make_async_copy(cp_hbm.at[src_v.at[b]], cps_v.at[b],
                              sems[b][0]).wait()
        for j in range(CH // 16):
            d16 = dst_v[b, pl.ds(j * 16, 16)]
            c16 = cps_v[b, pl.ds(j * 16, 16)]
            fidx_v[pl.ds(j * 16, 16)] = d16 * C + c16
        pltpu.sync_copy(ones_v, cnt_sh.at[fidx_v], add=True)
        pltpu.make_async_copy(feat_hbm.at[src_v.at[b]], rows_v.at[b],
                              sems[b][1]).wait()
        pltpu.sync_copy(rows_v.at[b], agg_sh.at[dst_v.at[b]], add=True)

    fire(0, 0)

    def body(k, _):
        fire(2 * k + 1, 1)
        drain(0)
        fire(2 * k + 2, 0)
        drain(1)
        return 0

    lax.fori_loop(0, (NCHUNK - 1) // 2, body, 0)
    drain(0)
    plsc.subcore_barrier()
    for j in range(8):
        pltpu.sync_copy(agg_sh.at[pl.ds(s * NSL + j * CH, CH), :], rows_v.at[0])
        pltpu.sync_copy(rows_v.at[0],
                        agg_out.at[pl.ds(c * N2 + s * NSL + j * CH, CH), :])
    for j in range(25):
        pltpu.sync_copy(cnt_sh.at[pl.ds(s * CSL + j * 1000, 1000)],
                        zcnt_v.at[pl.ds(0, 1000)])
        obase = pl.multiple_of(c * NCF + s * CSL + j * 1000, 8)
        pltpu.sync_copy(zcnt_v.at[pl.ds(0, 1000)],
                        cnt_out.at[pl.ds(obase, 1000)])


def _argmax_rows(x):
    m = jnp.max(x, axis=1, keepdims=True)
    ci = lax.broadcasted_iota(jnp.int32, x.shape, 1)
    return jnp.min(jnp.where(x == m, ci, x.shape[1]), axis=1, keepdims=True)


def _tc_pre_body(degp_ref, feat_ref, logits_ref, fs_ref, cp_ref,
                 norm_ref, deg_ref):
    indeg = degp_ref[0, 0] + degp_ref[1, 0]
    deg = jnp.maximum(indeg, 1.0)
    norm = 1.0 / jnp.sqrt(deg)
    deg_ref[0] = deg
    norm_ref[0] = norm
    fs_ref[...] = feat_ref[...] * norm
    cp_ref[0] = _argmax_rows(logits_ref[...])


def _tc_gate_body(cntp_ref, cp_ref, deg_ref, t1_ref, t2_ref, z_ref):
    counts = cntp_ref[0] + cntp_ref[1]
    deg = deg_ref[...]
    cp = cp_ref[...]
    ci = lax.broadcasted_iota(jnp.int32, counts.shape, 1)
    f1 = jnp.sum(jnp.where(ci == cp, counts, 0.0), axis=1,
                 keepdims=True) / deg
    present = jnp.sum(counts, axis=0, keepdims=True) > 0
    p = jnp.maximum(counts / deg, 1e-5)
    f2 = -jnp.sum(jnp.where(present, p * jnp.log(p), 0.0), axis=1,
                  keepdims=True)
    m1 = jnp.mean(f1)
    v1 = jnp.mean((f1 - m1) ** 2)
    nf1 = (f1 - m1) / jnp.sqrt(v1 + 1e-5)
    m2 = jnp.mean(f2)
    v2 = jnp.mean((f2 - m2) ** 2)
    nf2 = (f2 - m2) / jnp.sqrt(v2 + 1e-5)
    a = -(nf1 - t1_ref[0, 0])
    b = -(nf2 - t2_ref[0, 0])
    z_ref[...] = (1.0 / (1.0 + jnp.exp(-a))) * (1.0 / (1.0 + jnp.exp(-b)))


def _tc_update_body(z_ref, norm_ref, aggp_ref, fs_ref, wyt_ref, by_ref,
                    fnew_ref, fsn_ref, cpn_ref):
    agg = aggp_ref[0] + aggp_ref[1]
    z = z_ref[0]
    norm = norm_ref[0]
    fnew = z * (agg * norm) + fs_ref[...]
    fnew_ref[...] = fnew
    fsn = fnew * norm
    fsn_ref[...] = fsn
    logits = jnp.dot(fsn, wyt_ref[...],
                     preferred_element_type=jnp.float32) + by_ref[...]
    cpn_ref[0] = _argmax_rows(logits)


_GRID = 10
_BN = N // _GRID


def _wide_spec(width):
    return pl.BlockSpec((_BN, width), lambda i: (i, 0))


def _skinny_spec():
    return pl.BlockSpec((1, _BN, 1), lambda i: (i, 0, 0))


def _skinny_shape(dtype):
    return jax.ShapeDtypeStruct((_GRID, _BN, 1), dtype)


def _tc_pre(degp, feat, logits):
    return pl.pallas_call(
        _tc_pre_body,
        grid=(_GRID,),
        in_specs=[
            pl.BlockSpec((2, 1, _BN, 1), lambda i: (0, i, 0, 0)),
            _wide_spec(D),
            _wide_spec(C),
        ],
        out_specs=[_wide_spec(D), _skinny_spec(), _skinny_spec(),
                   _skinny_spec()],
        out_shape=[
            jax.ShapeDtypeStruct((N, D), jnp.float32),
            _skinny_shape(jnp.int32),
            _skinny_shape(jnp.float32),
            _skinny_shape(jnp.float32),
        ],
    )(degp, feat, logits)


def _tc_gate(cntp, cp, deg, t1, t2):
    return pl.pallas_call(
        _tc_gate_body,
        out_shape=jax.ShapeDtypeStruct((N, 1), jnp.float32),
    )(cntp, cp, deg, t1, t2)


def _tc_update(z, norm, aggp, fs, wyt, by2):
    return pl.pallas_call(
        _tc_update_body,
        grid=(_GRID,),
        in_specs=[
            _skinny_spec(),
            _skinny_spec(),
            pl.BlockSpec((2, _BN, D), lambda i: (0, i, 0)),
            _wide_spec(D),
            pl.BlockSpec((D, C), lambda i: (0, 0)),
            pl.BlockSpec((1, C), lambda i: (0, 0)),
        ],
        out_specs=[_wide_spec(D), _wide_spec(D), _skinny_spec()],
        out_shape=[
            jax.ShapeDtypeStruct((N, D), jnp.float32),
            jax.ShapeDtypeStruct((N, D), jnp.float32),
            _skinny_shape(jnp.int32),
        ],
    )(z, norm, aggp, fs, wyt, by2)


def kernel(feat, edge_index, logits, tau_1, tau_2, Wy, by):
    src = edge_index[0]
    dst = edge_index[1]
    wyt = Wy.T
    by2 = by.reshape(1, C)
    t1 = tau_1.reshape(1, 1)
    t2 = tau_2.reshape(1, 1)

    sc_degree, sc_edges = _sc_kernels()
    degp = sc_degree(dst)
    degp = degp.reshape(NC, N2)[:, :N].reshape(NC, _GRID, _BN, 1)
    fs, cp_b, norm_b, deg_b = _tc_pre(degp, feat, logits)
    deg = deg_b.reshape(N, 1)

    fnew = feat
    for _ in range(K):
        agg2, cnt2 = sc_edges(fs, cp_b.reshape(N), src, dst)
        cntp = cnt2.reshape(NC, N, C)
        aggp = agg2.reshape(NC, N2, D)[:, :N, :]
        z = _tc_gate(cntp, cp_b.reshape(N, 1), deg, t1, t2)
        fnew, fs, cp_b = _tc_update(z.reshape(_GRID, _BN, 1), norm_b,
                                    aggp, fs, wyt, by2)
    return fnew
